# Initial kernel scaffold; baseline (speedup 1.0000x reference)
import jax
import jax.numpy as jnp
from jax.experimental import pallas as pl


def kernel(h, norm, edge_index):
    raise NotImplementedError("write your pallas kernel here")


# speedup vs baseline: 8.3926x; 8.3926x over previous
import functools

import jax
import jax.numpy as jnp
from jax import lax
from jax.experimental import pallas as pl
from jax.experimental.pallas import tpu as pltpu
from jax.experimental.pallas import tpu_sc as plsc

N_NODES = 10000
N_EDGES = 320000
D = 128

NC = 2
NS = 16
NW = NC * NS

E_PER_TILE = N_EDGES // NW
K = 125
CH = E_PER_TILE // K
NPAD = 10240
ROWS_PER_TILE = NPAD // NS
ZROWS = 32


def _scale_kernel(h_ref, n_ref, o_ref):
    o_ref[...] = h_ref[...] * n_ref[...]


def _combine_kernel(h_ref, n_ref, p_ref, o_ref):
    nv = n_ref[...]
    hv = h_ref[...]
    o_ref[...] = hv + (hv * nv + p_ref[0] + p_ref[1]) * nv


def _sc_segment_sum(hn, src3d, dst3d):
    mesh = plsc.VectorSubcoreMesh(core_axis_name="c", subcore_axis_name="s")

    @functools.partial(
        pl.kernel,
        out_type=jax.ShapeDtypeStruct((NC, NPAD, D), jnp.float32),
        mesh=mesh,
        scratch_types=[
            pltpu.VMEM_SHARED((NPAD, D), jnp.float32),
            pltpu.VMEM((CH, K), jnp.int32),
            pltpu.VMEM((CH, K), jnp.int32),
            pltpu.VMEM((K, D), jnp.float32),
            pltpu.VMEM((ZROWS, D), jnp.float32),
            pltpu.SemaphoreType.DMA,
        ],
    )
    def seg_sum(hn_hbm, src_hbm, dst_hbm, out_hbm,
                acc, srcv, dstv, rows, zbuf, sem):
        c = lax.axis_index("c")
        s = lax.axis_index("s")
        wid = c * NS + s

        @pl.loop(0, ZROWS)
        def _(i):
            @pl.loop(0, D // 16)
            def _(j):
                zbuf[i, pl.ds(j * 16, 16)] = jnp.zeros((16,), jnp.float32)

        @pl.loop(0, ROWS_PER_TILE // ZROWS)
        def _(i):
            pltpu.sync_copy(
                zbuf, acc.at[pl.ds(s * ROWS_PER_TILE + i * ZROWS, ZROWS)])

        plsc.subcore_barrier()

        pltpu.sync_copy(src_hbm.at[wid], srcv)
        pltpu.sync_copy(dst_hbm.at[wid], dstv)

        @pl.loop(0, CH)
        def _(j):
            pltpu.async_copy(hn_hbm.at[srcv.at[j]], rows, sem).wait()
            pltpu.sync_copy(rows, acc.at[dstv.at[j]], add=True)

        plsc.subcore_barrier()

        pltpu.sync_copy(
            acc.at[pl.ds(s * ROWS_PER_TILE, ROWS_PER_TILE)],
            out_hbm.at[c, pl.ds(s * ROWS_PER_TILE, ROWS_PER_TILE)])

    return seg_sum(hn, src3d, dst3d)


@jax.jit
def kernel(h, norm, edge_index):
    src = edge_index[0].astype(jnp.int32).reshape(NW, CH, K)
    dst = edge_index[1].astype(jnp.int32).reshape(NW, CH, K)

    grid = 5
    rb = N_NODES // grid

    hn = pl.pallas_call(
        _scale_kernel,
        grid=(grid,),
        in_specs=[
            pl.BlockSpec((rb, D), lambda i: (i, 0)),
            pl.BlockSpec((rb, 1), lambda i: (i, 0)),
        ],
        out_specs=pl.BlockSpec((rb, D), lambda i: (i, 0)),
        out_shape=jax.ShapeDtypeStruct((N_NODES, D), jnp.float32),
    )(h, norm)

    partials = _sc_segment_sum(hn, src, dst)

    return pl.pallas_call(
        _combine_kernel,
        grid=(grid,),
        in_specs=[
            pl.BlockSpec((rb, D), lambda i: (i, 0)),
            pl.BlockSpec((rb, 1), lambda i: (i, 0)),
            pl.BlockSpec((NC, rb, D), lambda i: (0, i, 0)),
        ],
        out_specs=pl.BlockSpec((rb, D), lambda i: (i, 0)),
        out_shape=jax.ShapeDtypeStruct((N_NODES, D), jnp.float32),
    )(h, norm, partials)


# speedup vs baseline: 12.0276x; 1.4331x over previous
import functools

import jax
import jax.numpy as jnp
from jax import lax
from jax.experimental import pallas as pl
from jax.experimental.pallas import tpu as pltpu
from jax.experimental.pallas import tpu_sc as plsc

N_NODES = 10000
N_EDGES = 320000
D = 128

NC = 2
NS = 16
NW = NC * NS

E_PER_TILE = N_EDGES // NW
K = 125
CH = E_PER_TILE // K
G = 8
NG = CH // G
NPAD = 10240
ROWS_PER_TILE = NPAD // NS
ZROWS = 32


def _scale_kernel(h_ref, n_ref, o_ref):
    o_ref[...] = h_ref[...] * n_ref[...]


def _combine_kernel(h_ref, n_ref, p_ref, o_ref):
    nv = n_ref[...]
    hv = h_ref[...]
    o_ref[...] = hv + (hv * nv + p_ref[0] + p_ref[1]) * nv


def _sc_segment_sum(hn, src3d, dst3d):
    mesh = plsc.VectorSubcoreMesh(core_axis_name="c", subcore_axis_name="s")

    @functools.partial(
        pl.kernel,
        out_type=jax.ShapeDtypeStruct((NC, NPAD, D), jnp.float32),
        mesh=mesh,
        scratch_types=[
            pltpu.VMEM_SHARED((NPAD, D), jnp.float32),
            pltpu.VMEM((2, G, K), jnp.int32),
            pltpu.VMEM((2, G, K), jnp.int32),
            pltpu.VMEM((2, K, D), jnp.float32),
            pltpu.VMEM((ZROWS, D), jnp.float32),
            pltpu.SemaphoreType.DMA((2,)),
            pltpu.SemaphoreType.DMA,
        ],
    )
    def seg_sum(hn_hbm, src_hbm, dst_hbm, out_hbm,
                acc, srcv, dstv, rows, zbuf, gsem, isem):
        c = lax.axis_index("c")
        s = lax.axis_index("s")
        wid = c * NS + s

        def load_group(g, buf):
            sl = pl.ds(g * G, G)
            return (pltpu.async_copy(src_hbm.at[wid, sl], srcv.at[buf], isem),
                    pltpu.async_copy(dst_hbm.at[wid, sl], dstv.at[buf], isem))

        def wait_group(g, buf):
            sl = pl.ds(g * G, G)
            pltpu.make_async_copy(src_hbm.at[wid, sl], srcv.at[buf],
                                  isem).wait()
            pltpu.make_async_copy(dst_hbm.at[wid, sl], dstv.at[buf],
                                  isem).wait()

        load_group(0, 0)

        @pl.loop(0, ZROWS)
        def _(i):
            @pl.loop(0, D // 16)
            def _(j):
                zbuf[i, pl.ds(j * 16, 16)] = jnp.zeros((16,), jnp.float32)

        @pl.loop(0, ROWS_PER_TILE // ZROWS)
        def _(i):
            pltpu.sync_copy(
                zbuf, acc.at[pl.ds(s * ROWS_PER_TILE + i * ZROWS, ZROWS)])

        plsc.subcore_barrier()
        wait_group(0, 0)

        pltpu.async_copy(hn_hbm.at[srcv.at[0, 0]], rows.at[0], gsem.at[0])

        @pl.loop(0, NG)
        def _(g):
            bg = lax.rem(g, 2)
            nbg = 1 - bg

            @pl.when(g + 1 < NG)
            def _():
                load_group(g + 1, nbg)

            @pl.loop(0, G)
            def _(r):
                j = g * G + r
                b = lax.rem(j, 2)
                nb = 1 - b

                @pl.when(r + 1 < G)
                def _():
                    pltpu.async_copy(hn_hbm.at[srcv.at[bg, r + 1]],
                                     rows.at[nb], gsem.at[nb])

                @pl.when(jnp.logical_and(r + 1 == G, g + 1 < NG))
                def _():
                    wait_group(g + 1, nbg)
                    pltpu.async_copy(hn_hbm.at[srcv.at[nbg, 0]],
                                     rows.at[nb], gsem.at[nb])

                pltpu.make_async_copy(hn_hbm.at[srcv.at[bg, r]], rows.at[b],
                                      gsem.at[b]).wait()
                pltpu.sync_copy(rows.at[b], acc.at[dstv.at[bg, r]], add=True)

        plsc.subcore_barrier()

        pltpu.sync_copy(
            acc.at[pl.ds(s * ROWS_PER_TILE, ROWS_PER_TILE)],
            out_hbm.at[c, pl.ds(s * ROWS_PER_TILE, ROWS_PER_TILE)])

    return seg_sum(hn, src3d, dst3d)


@jax.jit
def kernel(h, norm, edge_index):
    src = edge_index[0].astype(jnp.int32).reshape(NW, CH, K)
    dst = edge_index[1].astype(jnp.int32).reshape(NW, CH, K)

    grid = 5
    rb = N_NODES // grid

    hn = pl.pallas_call(
        _scale_kernel,
        grid=(grid,),
        in_specs=[
            pl.BlockSpec((rb, D), lambda i: (i, 0)),
            pl.BlockSpec((rb, 1), lambda i: (i, 0)),
        ],
        out_specs=pl.BlockSpec((rb, D), lambda i: (i, 0)),
        out_shape=jax.ShapeDtypeStruct((N_NODES, D), jnp.float32),
    )(h, norm)

    partials = _sc_segment_sum(hn, src, dst)

    return pl.pallas_call(
        _combine_kernel,
        grid=(grid,),
        in_specs=[
            pl.BlockSpec((rb, D), lambda i: (i, 0)),
            pl.BlockSpec((rb, 1), lambda i: (i, 0)),
            pl.BlockSpec((NC, rb, D), lambda i: (0, i, 0)),
        ],
        out_specs=pl.BlockSpec((rb, D), lambda i: (i, 0)),
        out_shape=jax.ShapeDtypeStruct((N_NODES, D), jnp.float32),
    )(h, norm, partials)


# speedup vs baseline: 12.5780x; 1.0458x over previous
import functools

import jax
import jax.numpy as jnp
from jax import lax
from jax.experimental import pallas as pl
from jax.experimental.pallas import tpu as pltpu
from jax.experimental.pallas import tpu_sc as plsc

N_NODES = 10000
N_EDGES = 320000
D = 128

NC = 2
NS = 16
NW = NC * NS

E_PER_TILE = N_EDGES // NW
K = 125
CH = E_PER_TILE // K
G = 8
NG = CH // G
NPAD = 10240
ROWS_PER_TILE = NPAD // NS
ZROWS = 32


def _scale_kernel(h_ref, n_ref, o_ref):
    o_ref[...] = h_ref[...] * n_ref[...]


def _combine_kernel(h_ref, n_ref, p_ref, o_ref):
    nv = n_ref[...]
    hv = h_ref[...]
    o_ref[...] = hv + (hv * nv + p_ref[0] + p_ref[1]) * nv


def _sc_segment_sum(hn, eidx):
    mesh = plsc.VectorSubcoreMesh(core_axis_name="c", subcore_axis_name="s")

    @functools.partial(
        pl.kernel,
        out_type=jax.ShapeDtypeStruct((NC, NPAD, D), jnp.float32),
        mesh=mesh,
        scratch_types=[
            pltpu.VMEM_SHARED((NPAD, D), jnp.float32),
            pltpu.VMEM((2, G, K), jnp.int32),
            pltpu.VMEM((2, G, K), jnp.int32),
            pltpu.VMEM((2, K, D), jnp.float32),
            pltpu.VMEM((ZROWS, D), jnp.float32),
            pltpu.SemaphoreType.DMA((2,)),
            pltpu.SemaphoreType.DMA,
        ],
    )
    def seg_sum(hn_hbm, eidx_hbm, out_hbm,
                acc, srcv, dstv, rows, zbuf, gsem, isem):
        c = lax.axis_index("c")
        s = lax.axis_index("s")
        wid = c * NS + s

        def load_group(g, buf):
            sl = pl.ds(wid * CH + g * G, G)
            return (pltpu.async_copy(eidx_hbm.at[0, sl], srcv.at[buf], isem),
                    pltpu.async_copy(eidx_hbm.at[1, sl], dstv.at[buf], isem))

        def wait_group(g, buf):
            sl = pl.ds(wid * CH + g * G, G)
            pltpu.make_async_copy(eidx_hbm.at[0, sl], srcv.at[buf],
                                  isem).wait()
            pltpu.make_async_copy(eidx_hbm.at[1, sl], dstv.at[buf],
                                  isem).wait()

        load_group(0, 0)

        @pl.loop(0, ZROWS)
        def _(i):
            @pl.loop(0, D // 16)
            def _(j):
                zbuf[i, pl.ds(j * 16, 16)] = jnp.zeros((16,), jnp.float32)

        @pl.loop(0, ROWS_PER_TILE // ZROWS)
        def _(i):
            pltpu.sync_copy(
                zbuf, acc.at[pl.ds(s * ROWS_PER_TILE + i * ZROWS, ZROWS)])

        plsc.subcore_barrier()
        wait_group(0, 0)

        pltpu.async_copy(hn_hbm.at[srcv.at[0, 0]], rows.at[0], gsem.at[0])

        @pl.loop(0, NG)
        def _(g):
            bg = lax.rem(g, 2)
            nbg = 1 - bg

            @pl.when(g + 1 < NG)
            def _():
                load_group(g + 1, nbg)

            @pl.loop(0, G)
            def _(r):
                j = g * G + r
                b = lax.rem(j, 2)
                nb = 1 - b

                @pl.when(r + 1 < G)
                def _():
                    pltpu.async_copy(hn_hbm.at[srcv.at[bg, r + 1]],
                                     rows.at[nb], gsem.at[nb])

                @pl.when(jnp.logical_and(r + 1 == G, g + 1 < NG))
                def _():
                    wait_group(g + 1, nbg)
                    pltpu.async_copy(hn_hbm.at[srcv.at[nbg, 0]],
                                     rows.at[nb], gsem.at[nb])

                pltpu.make_async_copy(hn_hbm.at[srcv.at[bg, r]], rows.at[b],
                                      gsem.at[b]).wait()
                pltpu.sync_copy(rows.at[b], acc.at[dstv.at[bg, r]], add=True)

        plsc.subcore_barrier()

        pltpu.sync_copy(
            acc.at[pl.ds(s * ROWS_PER_TILE, ROWS_PER_TILE)],
            out_hbm.at[c, pl.ds(s * ROWS_PER_TILE, ROWS_PER_TILE)])

    return seg_sum(hn, eidx)


@jax.jit
def kernel(h, norm, edge_index):
    eidx = edge_index.astype(jnp.int32).reshape(2, NW * CH, K)

    grid = 10
    rb = N_NODES // grid

    hn = pl.pallas_call(
        _scale_kernel,
        grid=(grid,),
        in_specs=[
            pl.BlockSpec((rb, D), lambda i: (i, 0)),
            pl.BlockSpec((rb, 1), lambda i: (i, 0)),
        ],
        out_specs=pl.BlockSpec((rb, D), lambda i: (i, 0)),
        out_shape=jax.ShapeDtypeStruct((N_NODES, D), jnp.float32),
    )(h, norm)

    partials = _sc_segment_sum(hn, eidx)

    return pl.pallas_call(
        _combine_kernel,
        grid=(grid,),
        in_specs=[
            pl.BlockSpec((rb, D), lambda i: (i, 0)),
            pl.BlockSpec((rb, 1), lambda i: (i, 0)),
            pl.BlockSpec((NC, rb, D), lambda i: (0, i, 0)),
        ],
        out_specs=pl.BlockSpec((rb, D), lambda i: (i, 0)),
        out_shape=jax.ShapeDtypeStruct((N_NODES, D), jnp.float32),
    )(h, norm, partials)


# speedup vs baseline: 12.8905x; 1.0248x over previous
import functools

import jax
import jax.numpy as jnp
from jax import lax
from jax.experimental import pallas as pl
from jax.experimental.pallas import tpu as pltpu
from jax.experimental.pallas import tpu_sc as plsc

N_NODES = 10000
N_EDGES = 320000
D = 128

NC = 2
NS = 16
NW = NC * NS

K = 128
CH = 80
G = 8
NG = CH // G
EPAD = NW * CH * K
NPAD = 10240
ROWS_PER_TILE = NPAD // NS
ZROWS = 32


def _scale_kernel(h_ref, n_ref, o_ref):
    o_ref[...] = h_ref[...] * n_ref[...]


def _combine_kernel(h_ref, n_ref, p_ref, o_ref):
    nv = n_ref[...]
    hv = h_ref[...]
    o_ref[...] = hv + (hv * nv + p_ref[0] + p_ref[1]) * nv


def _sc_segment_sum(hn, eidx):
    mesh = plsc.VectorSubcoreMesh(core_axis_name="c", subcore_axis_name="s")

    @functools.partial(
        pl.kernel,
        out_type=jax.ShapeDtypeStruct((NC, N_NODES, D), jnp.float32),
        mesh=mesh,
        scratch_types=[
            pltpu.VMEM_SHARED((NPAD, D), jnp.float32),
            pltpu.VMEM((2, G, K), jnp.int32),
            pltpu.VMEM((2, G, K), jnp.int32),
            pltpu.VMEM((2, K, D), jnp.float32),
            pltpu.VMEM((ZROWS, D), jnp.float32),
            pltpu.SemaphoreType.DMA((2,)),
            pltpu.SemaphoreType.DMA,
        ],
    )
    def seg_sum(hn_hbm, eidx_hbm, out_hbm,
                acc, srcv, dstv, rows, zbuf, gsem, isem):
        c = lax.axis_index("c")
        s = lax.axis_index("s")
        wid = c * NS + s

        def load_group(g, buf):
            sl = pl.ds(wid * CH + g * G, G)
            return (pltpu.async_copy(eidx_hbm.at[0, sl], srcv.at[buf], isem),
                    pltpu.async_copy(eidx_hbm.at[1, sl], dstv.at[buf], isem))

        def wait_group(g, buf):
            sl = pl.ds(wid * CH + g * G, G)
            pltpu.make_async_copy(eidx_hbm.at[0, sl], srcv.at[buf],
                                  isem).wait()
            pltpu.make_async_copy(eidx_hbm.at[1, sl], dstv.at[buf],
                                  isem).wait()

        load_group(0, 0)

        @pl.loop(0, ZROWS)
        def _(i):
            @pl.loop(0, D // 16)
            def _(j):
                zbuf[i, pl.ds(j * 16, 16)] = jnp.zeros((16,), jnp.float32)

        @pl.loop(0, ROWS_PER_TILE // ZROWS)
        def _(i):
            pltpu.sync_copy(
                zbuf, acc.at[pl.ds(s * ROWS_PER_TILE + i * ZROWS, ZROWS)])

        plsc.subcore_barrier()
        wait_group(0, 0)

        pltpu.async_copy(hn_hbm.at[srcv.at[0, 0]], rows.at[0], gsem.at[0])

        @pl.loop(0, NG)
        def _(g):
            bg = lax.rem(g, 2)
            nbg = 1 - bg

            @pl.when(g + 1 < NG)
            def _():
                load_group(g + 1, nbg)

            @pl.loop(0, G)
            def _(r):
                j = g * G + r
                b = lax.rem(j, 2)
                nb = 1 - b

                @pl.when(r + 1 < G)
                def _():
                    pltpu.async_copy(hn_hbm.at[srcv.at[bg, r + 1]],
                                     rows.at[nb], gsem.at[nb])

                @pl.when(jnp.logical_and(r + 1 == G, g + 1 < NG))
                def _():
                    wait_group(g + 1, nbg)
                    pltpu.async_copy(hn_hbm.at[srcv.at[nbg, 0]],
                                     rows.at[nb], gsem.at[nb])

                pltpu.make_async_copy(hn_hbm.at[srcv.at[bg, r]], rows.at[b],
                                      gsem.at[b]).wait()
                pltpu.sync_copy(rows.at[b], acc.at[dstv.at[bg, r]], add=True)

        plsc.subcore_barrier()

        @pl.when(s < NS - 1)
        def _():
            pltpu.sync_copy(
                acc.at[pl.ds(s * ROWS_PER_TILE, ROWS_PER_TILE)],
                out_hbm.at[c, pl.ds(s * ROWS_PER_TILE, ROWS_PER_TILE)])

        @pl.when(s == NS - 1)
        def _():
            last = N_NODES - (NS - 1) * ROWS_PER_TILE
            pltpu.sync_copy(
                acc.at[pl.ds((NS - 1) * ROWS_PER_TILE, last)],
                out_hbm.at[c, pl.ds((NS - 1) * ROWS_PER_TILE, last)])

    return seg_sum(hn, eidx)


@jax.jit
def kernel(h, norm, edge_index):
    npad_e = EPAD - N_EDGES
    pad_iota = jnp.arange(npad_e, dtype=jnp.int32)
    pad = jnp.stack([pad_iota % N_NODES, N_NODES + pad_iota % (NPAD - N_NODES)])
    eidx = jnp.concatenate([edge_index.astype(jnp.int32), pad], axis=1)
    eidx = eidx.reshape(2, NW * CH, K)

    hn = pl.pallas_call(
        _scale_kernel,
        out_shape=jax.ShapeDtypeStruct((N_NODES, D), jnp.float32),
    )(h, norm)

    partials = _sc_segment_sum(hn, eidx)

    grid = 5
    rb = N_NODES // grid
    return pl.pallas_call(
        _combine_kernel,
        grid=(grid,),
        in_specs=[
            pl.BlockSpec((rb, D), lambda i: (i, 0)),
            pl.BlockSpec((rb, 1), lambda i: (i, 0)),
            pl.BlockSpec((NC, rb, D), lambda i: (0, i, 0)),
        ],
        out_specs=pl.BlockSpec((rb, D), lambda i: (i, 0)),
        out_shape=jax.ShapeDtypeStruct((N_NODES, D), jnp.float32),
    )(h, norm, partials)


# speedup vs baseline: 13.9476x; 1.0820x over previous
import functools

import jax
import jax.numpy as jnp
from jax import lax
from jax.experimental import pallas as pl
from jax.experimental.pallas import tpu as pltpu
from jax.experimental.pallas import tpu_sc as plsc

N_NODES = 10000
N_EDGES = 320000
D = 128

NC = 2
NS = 16
NW = NC * NS

K = 64
NB = 4
LOOK = NB - 1
CH = 160
G = 8
NG = CH // G
EPAD = NW * CH * K
NPAD = 10240
ROWS_PER_TILE = NPAD // NS
ZROWS = 32


def _scale_kernel(h_ref, n_ref, o_ref):
    o_ref[...] = h_ref[...] * n_ref[...]


def _combine_kernel(h_ref, n_ref, p_ref, o_ref):
    nv = n_ref[...]
    hv = h_ref[...]
    o_ref[...] = hv + (hv * nv + p_ref[0] + p_ref[1]) * nv


def _sc_segment_sum(hn, eidx):
    mesh = plsc.VectorSubcoreMesh(core_axis_name="c", subcore_axis_name="s")

    @functools.partial(
        pl.kernel,
        out_type=jax.ShapeDtypeStruct((NC, N_NODES, D), jnp.float32),
        mesh=mesh,
        scratch_types=[
            pltpu.VMEM_SHARED((NPAD, D), jnp.float32),
            pltpu.VMEM((2, G, K), jnp.int32),
            pltpu.VMEM((2, G, K), jnp.int32),
            pltpu.VMEM((NB, K, D), jnp.float32),
            pltpu.VMEM((ZROWS, D), jnp.float32),
            pltpu.SemaphoreType.DMA((NB,)),
            pltpu.SemaphoreType.DMA((NB,)),
            pltpu.SemaphoreType.DMA,
        ],
    )
    def seg_sum(hn_hbm, eidx_hbm, out_hbm,
                acc, srcv, dstv, rows, zbuf, gsem, ssem, isem):
        c = lax.axis_index("c")
        s = lax.axis_index("s")
        wid = c * NS + s

        def load_group(g, buf):
            sl = pl.ds(wid * CH + g * G, G)
            return (pltpu.async_copy(eidx_hbm.at[0, sl], srcv.at[buf], isem),
                    pltpu.async_copy(eidx_hbm.at[1, sl], dstv.at[buf], isem))

        def wait_group(g, buf):
            sl = pl.ds(wid * CH + g * G, G)
            pltpu.make_async_copy(eidx_hbm.at[0, sl], srcv.at[buf],
                                  isem).wait()
            pltpu.make_async_copy(eidx_hbm.at[1, sl], dstv.at[buf],
                                  isem).wait()

        def issue_gather(gbuf, r, b):
            pltpu.async_copy(hn_hbm.at[srcv.at[gbuf, r]], rows.at[b],
                             gsem.at[b])

        def wait_gather(gbuf, r, b):
            pltpu.make_async_copy(hn_hbm.at[srcv.at[gbuf, r]], rows.at[b],
                                  gsem.at[b]).wait()

        def issue_scatter(gbuf, r, b):
            pltpu.async_copy(rows.at[b], acc.at[dstv.at[gbuf, r]],
                             ssem.at[b], add=True)

        def wait_scatter(gbuf, r, b):
            pltpu.make_async_copy(rows.at[b], acc.at[dstv.at[gbuf, r]],
                                  ssem.at[b]).wait()

        load_group(0, 0)

        @pl.loop(0, ZROWS)
        def _(i):
            @pl.loop(0, D // 16)
            def _(j):
                zbuf[i, pl.ds(j * 16, 16)] = jnp.zeros((16,), jnp.float32)

        @pl.loop(0, ROWS_PER_TILE // ZROWS)
        def _(i):
            pltpu.sync_copy(
                zbuf, acc.at[pl.ds(s * ROWS_PER_TILE + i * ZROWS, ZROWS)])

        plsc.subcore_barrier()
        wait_group(0, 0)

        @pl.loop(0, LOOK)
        def _(r):
            issue_gather(0, r, r)

        @pl.loop(0, NG)
        def _(g):
            bg = lax.rem(g, 2)
            nbg = 1 - bg

            @pl.when(g + 1 < NG)
            def _():
                load_group(g + 1, nbg)

            @pl.loop(0, G)
            def _(r):
                j = g * G + r
                b = lax.rem(j, NB)
                lb = lax.rem(j + LOOK, NB)

                @pl.when(jnp.logical_and(r == G - LOOK, g + 1 < NG))
                def _():
                    wait_group(g + 1, nbg)

                @pl.when(jnp.logical_and(j >= 1, j + LOOK < CH))
                def _():
                    pj = j - 1
                    pg = pj // G
                    wait_scatter(lax.rem(pg, 2), pj - pg * G, lb)

                @pl.when(r + LOOK < G)
                def _():
                    issue_gather(bg, r + LOOK, lb)

                @pl.when(jnp.logical_and(r + LOOK >= G, g + 1 < NG))
                def _():
                    issue_gather(nbg, r + LOOK - G, lb)

                wait_gather(bg, r, b)
                issue_scatter(bg, r, b)

        @pl.loop(CH - NB, CH)
        def _(j):
            g = j // G
            wait_scatter(lax.rem(g, 2), j - g * G, lax.rem(j, NB))

        plsc.subcore_barrier()

        @pl.when(s < NS - 1)
        def _():
            pltpu.sync_copy(
                acc.at[pl.ds(s * ROWS_PER_TILE, ROWS_PER_TILE)],
                out_hbm.at[c, pl.ds(s * ROWS_PER_TILE, ROWS_PER_TILE)])

        @pl.when(s == NS - 1)
        def _():
            last = N_NODES - (NS - 1) * ROWS_PER_TILE
            pltpu.sync_copy(
                acc.at[pl.ds((NS - 1) * ROWS_PER_TILE, last)],
                out_hbm.at[c, pl.ds((NS - 1) * ROWS_PER_TILE, last)])

    return seg_sum(hn, eidx)


@jax.jit
def kernel(h, norm, edge_index):
    npad_e = EPAD - N_EDGES
    pad_iota = jnp.arange(npad_e, dtype=jnp.int32)
    pad = jnp.stack([pad_iota % N_NODES, N_NODES + pad_iota % (NPAD - N_NODES)])
    eidx = jnp.concatenate([edge_index.astype(jnp.int32), pad], axis=1)
    eidx = eidx.reshape(2, NW * CH, K)

    hn = pl.pallas_call(
        _scale_kernel,
        out_shape=jax.ShapeDtypeStruct((N_NODES, D), jnp.float32),
    )(h, norm)

    partials = _sc_segment_sum(hn, eidx)

    grid = 5
    rb = N_NODES // grid
    return pl.pallas_call(
        _combine_kernel,
        grid=(grid,),
        in_specs=[
            pl.BlockSpec((rb, D), lambda i: (i, 0)),
            pl.BlockSpec((rb, 1), lambda i: (i, 0)),
            pl.BlockSpec((NC, rb, D), lambda i: (0, i, 0)),
        ],
        out_specs=pl.BlockSpec((rb, D), lambda i: (i, 0)),
        out_shape=jax.ShapeDtypeStruct((N_NODES, D), jnp.float32),
    )(h, norm, partials)
